# Initial kernel scaffold; baseline (speedup 1.0000x reference)
import jax
import jax.numpy as jnp
from jax.experimental import pallas as pl


def kernel(x, x_complete, W0, b0, g0, beta0, W1, b1, g1, beta1):
    raise NotImplementedError("write your pallas kernel here")


# speedup vs baseline: 127.6952x; 127.6952x over previous
import functools

import numpy as np
import jax
import jax.numpy as jnp
from jax import lax
from jax.experimental import pallas as pl
from jax.experimental.pallas import tpu as pltpu
from jax.experimental.pallas import tpu_sc as plsc

_B, _N, _S, _K = 4, 8192, 1024, 32
_R2 = np.float32(0.04)
_CIN = 64
_D1, _D2 = 32, 64
_EPS = np.float32(1e-5)
_DF = 128
_ROWS = _B * _S * _K
_SBLK = 64
_RBLK = _SBLK * _K
_NBLK = (_B * _S) // _SBLK
_GCH = 128


def _fps_kernel(x0_ref, x1_ref, x2_ref, c0_ref, c1_ref, c2_ref, dist_ref):
    lane = lax.broadcasted_iota(jnp.int32, (_B, _N), 1)
    lane_s = lax.broadcasted_iota(jnp.int32, (_B, _S), 1)
    dist_ref[...] = jnp.full((_B, _N), 1e10, jnp.float32)

    def body(i, far):
        x0 = x0_ref[...]
        x1 = x1_ref[...]
        x2 = x2_ref[...]
        eq = lane == far
        c0 = jnp.sum(jnp.where(eq, x0, 0.0), axis=1, keepdims=True)
        c1 = jnp.sum(jnp.where(eq, x1, 0.0), axis=1, keepdims=True)
        c2 = jnp.sum(jnp.where(eq, x2, 0.0), axis=1, keepdims=True)
        sel = lane_s == i
        c0_ref[...] = jnp.where(sel, c0, c0_ref[...])
        c1_ref[...] = jnp.where(sel, c1, c1_ref[...])
        c2_ref[...] = jnp.where(sel, c2, c2_ref[...])
        d0 = x0 - c0
        d1 = x1 - c1
        d2 = x2 - c2
        d = (d0 * d0 + d1 * d1) + d2 * d2
        nd = jnp.minimum(dist_ref[...], d)
        dist_ref[...] = nd
        m = jnp.max(nd, axis=1, keepdims=True)
        return jnp.min(jnp.where(nd == m, lane, _N), axis=1, keepdims=True)

    lax.fori_loop(0, _S, body, jnp.zeros((_B, 1), jnp.int32))


def _fps(x0, x1, x2):
    return pl.pallas_call(
        _fps_kernel,
        out_shape=[jax.ShapeDtypeStruct((_B, _S), jnp.float32)] * 3,
        scratch_shapes=[pltpu.VMEM((_B, _N), jnp.float32)],
    )(x0, x1, x2)


def _bq_kernel(x0_ref, x1_ref, x2_ref, c0_ref, c1_ref, c2_ref,
               idx_ref, gidx_ref):
    b = pl.program_id(0)
    x0 = x0_ref[0]
    x1 = x1_ref[0]
    x2 = x2_ref[0]
    c0 = c0_ref[...]
    c1 = c1_ref[...]
    c2 = c2_ref[...]
    xn = (x0 * x0 + x1 * x1) + x2 * x2
    cn = (c0 * c0 + c1 * c1) + c2 * c2
    x0b = x0.astype(jnp.bfloat16).astype(jnp.float32)
    x1b = x1.astype(jnp.bfloat16).astype(jnp.float32)
    x2b = x2.astype(jnp.bfloat16).astype(jnp.float32)
    c0b = c0.astype(jnp.bfloat16).astype(jnp.float32)
    c1b = c1.astype(jnp.bfloat16).astype(jnp.float32)
    c2b = c2.astype(jnp.bfloat16).astype(jnp.float32)
    dot = (c0b * x0b + c1b * x1b) + c2b * x2b
    d2 = (cn + xn) - 2.0 * dot
    pos = (d2 < _R2).astype(jnp.float32)
    lanei = lax.broadcasted_iota(jnp.int32, (_SBLK, _N), 1)
    shift = 1
    while shift < _N:
        rolled = pltpu.roll(pos, shift, 1)
        pos = pos + jnp.where(lanei >= shift, rolled, 0.0)
        shift *= 2
    cols = []
    for j in range(1, _K + 1):
        cols.append(jnp.sum((pos < j).astype(jnp.float32), axis=1,
                            keepdims=True))
    cnt = jnp.concatenate(cols, axis=1)
    iv = cnt.astype(jnp.int32)
    idxv = jnp.where(iv >= _N, -1, iv)
    idx_ref[...] = idxv[None]
    gidx_ref[...] = (jnp.maximum(idxv, 0) + b * _N)[None]


def _bq(x0, x1, x2, c0t, c1t, c2t):
    return pl.pallas_call(
        _bq_kernel,
        grid=(_B, _S // _SBLK),
        in_specs=[pl.BlockSpec((1, 1, _N), lambda b, s: (b, 0, 0))] * 3
        + [pl.BlockSpec((_SBLK, 1),
                        lambda b, s: (b * (_S // _SBLK) + s, 0))] * 3,
        out_specs=[pl.BlockSpec((1, _SBLK, _K), lambda b, s: (b, s, 0))] * 2,
        out_shape=[jax.ShapeDtypeStruct((_B, _S, _K), jnp.int32)] * 2,
        compiler_params=pltpu.CompilerParams(
            dimension_semantics=("arbitrary", "arbitrary")),
    )(x0, x1, x2, c0t, c1t, c2t)


def _sc_gather(tab, gidx):
    nw = 32
    bpw = _ROWS // nw
    mesh = plsc.VectorSubcoreMesh(core_axis_name="c", subcore_axis_name="s")

    @functools.partial(
        pl.kernel, mesh=mesh,
        out_type=jax.ShapeDtypeStruct((_ROWS, _DF), jnp.float32),
        scratch_types=[pltpu.VMEM((_GCH,), jnp.int32),
                       pltpu.VMEM((_GCH, _DF), jnp.float32),
                       pltpu.SemaphoreType.DMA],
    )
    def k(tab_hbm, idx_hbm, out_hbm, idx_v, rows_v, sem):
        wid = lax.axis_index("s") * 2 + lax.axis_index("c")
        base = wid * bpw

        @pl.loop(0, bpw // _GCH)
        def _(c):
            b0 = base + c * _GCH
            pltpu.sync_copy(idx_hbm.at[pl.ds(b0, _GCH)], idx_v)
            pltpu.async_copy(tab_hbm.at[idx_v], rows_v, sem).wait()
            pltpu.sync_copy(rows_v, out_hbm.at[pl.ds(b0, _GCH)])

    return k(tab, gidx)


def _mlp1_kernel(f_ref, c0_ref, c1_ref, c2_ref, w_ref, b_ref,
                 h_ref, st_ref, acc_ref):
    i = pl.program_id(0)
    f = f_ref[...]
    w = w_ref[...]
    h = jnp.dot(f, w, preferred_element_type=jnp.float32) + b_ref[...]
    corr = (c0_ref[...] * w[_CIN:_CIN + 1, :]
            + c1_ref[...] * w[_CIN + 1:_CIN + 2, :]
            + c2_ref[...] * w[_CIN + 2:_CIN + 3, :])
    hp = (h.reshape(_SBLK, _K, _D1) - corr[:, None, :]).reshape(_RBLK, _D1)
    h_ref[...] = hp
    s = jnp.sum(hp, axis=0, keepdims=True)
    ss = jnp.sum(hp * hp, axis=0, keepdims=True)

    @pl.when(i == 0)
    def _():
        acc_ref[...] = jnp.zeros((2, _D1), jnp.float32)

    acc_ref[...] += jnp.concatenate([s, ss], axis=0)
    st_ref[...] = acc_ref[...]


def _mlp1(feat, c0t, c1t, c2t, w0p, b0):
    return pl.pallas_call(
        _mlp1_kernel,
        grid=(_NBLK,),
        in_specs=[pl.BlockSpec((_RBLK, _DF), lambda i: (i, 0))]
        + [pl.BlockSpec((_SBLK, 1), lambda i: (i, 0))] * 3
        + [pl.BlockSpec((_DF, _D1), lambda i: (0, 0)),
           pl.BlockSpec((1, _D1), lambda i: (0, 0))],
        out_specs=[pl.BlockSpec((_RBLK, _D1), lambda i: (i, 0)),
                   pl.BlockSpec((2, _D1), lambda i: (0, 0))],
        out_shape=[jax.ShapeDtypeStruct((_ROWS, _D1), jnp.float32),
                   jax.ShapeDtypeStruct((2, _D1), jnp.float32)],
        scratch_shapes=[pltpu.VMEM((2, _D1), jnp.float32)],
        compiler_params=pltpu.CompilerParams(
            dimension_semantics=("arbitrary",)),
    )(feat, c0t, c1t, c2t, w0p, b0)


def _mlp2_kernel(h_ref, st_ref, w_ref, b_ref, g_ref, be_ref,
                 h2_ref, st2_ref, acc_ref):
    i = pl.program_id(0)
    st = st_ref[...]
    m = st[0:1, :] * (1.0 / _ROWS)
    v = st[1:2, :] * (1.0 / _ROWS) - m * m
    sc = g_ref[...] / jnp.sqrt(v + _EPS)
    sh = be_ref[...] - m * sc
    h1 = jnp.maximum(h_ref[...] * sc + sh, 0.0)
    h2 = jnp.dot(h1, w_ref[...], preferred_element_type=jnp.float32) + b_ref[...]
    h2_ref[...] = h2
    s = jnp.sum(h2, axis=0, keepdims=True)
    ss = jnp.sum(h2 * h2, axis=0, keepdims=True)

    @pl.when(i == 0)
    def _():
        acc_ref[...] = jnp.zeros((2, _D2), jnp.float32)

    acc_ref[...] += jnp.concatenate([s, ss], axis=0)
    st2_ref[...] = acc_ref[...]


def _mlp2(h1, st1, w1t, b1, g0, beta0):
    return pl.pallas_call(
        _mlp2_kernel,
        grid=(_NBLK,),
        in_specs=[pl.BlockSpec((_RBLK, _D1), lambda i: (i, 0)),
                  pl.BlockSpec((2, _D1), lambda i: (0, 0)),
                  pl.BlockSpec((_D1, _D2), lambda i: (0, 0)),
                  pl.BlockSpec((1, _D2), lambda i: (0, 0)),
                  pl.BlockSpec((1, _D1), lambda i: (0, 0)),
                  pl.BlockSpec((1, _D1), lambda i: (0, 0))],
        out_specs=[pl.BlockSpec((_RBLK, _D2), lambda i: (i, 0)),
                   pl.BlockSpec((2, _D2), lambda i: (0, 0))],
        out_shape=[jax.ShapeDtypeStruct((_ROWS, _D2), jnp.float32),
                   jax.ShapeDtypeStruct((2, _D2), jnp.float32)],
        scratch_shapes=[pltpu.VMEM((2, _D2), jnp.float32)],
        compiler_params=pltpu.CompilerParams(
            dimension_semantics=("arbitrary",)),
    )(h1, st1, w1t, b1, g0, beta0)


def _pool_kernel(h2_ref, st_ref, g_ref, be_ref, idx_ref, o_ref):
    st = st_ref[...]
    m = st[0:1, :] * (1.0 / _ROWS)
    v = st[1:2, :] * (1.0 / _ROWS) - m * m
    sc = g_ref[...] / jnp.sqrt(v + _EPS)
    sh = be_ref[...] - m * sc
    h = jnp.maximum(h2_ref[...] * sc + sh, 0.0)
    msk = idx_ref[...] == -1
    hm = jnp.where(msk, -jnp.inf, h)
    o_ref[...] = jnp.max(hm.reshape(_SBLK, _K, _D2), axis=1)


def _pool(h2, st2, g1, beta1, idx2):
    return pl.pallas_call(
        _pool_kernel,
        grid=(_NBLK,),
        in_specs=[pl.BlockSpec((_RBLK, _D2), lambda i: (i, 0)),
                  pl.BlockSpec((2, _D2), lambda i: (0, 0)),
                  pl.BlockSpec((1, _D2), lambda i: (0, 0)),
                  pl.BlockSpec((1, _D2), lambda i: (0, 0)),
                  pl.BlockSpec((_RBLK, 1), lambda i: (i, 0))],
        out_specs=[pl.BlockSpec((_SBLK, _D2), lambda i: (i, 0))],
        out_shape=[jax.ShapeDtypeStruct((_B * _S, _D2), jnp.float32)],
        compiler_params=pltpu.CompilerParams(
            dimension_semantics=("arbitrary",)),
    )(h2, st2, g1, beta1, idx2)


def kernel(x, x_complete, W0, b0, g0, beta0, W1, b1, g1, beta1):
    x0 = x[:, :, 0]
    x1 = x[:, :, 1]
    x2 = x[:, :, 2]
    c0, c1, c2 = _fps(x0, x1, x2)
    centroids = jnp.stack([c0, c1, c2], axis=-1)
    c0t = c0.reshape(_B * _S, 1)
    c1t = c1.reshape(_B * _S, 1)
    c2t = c2.reshape(_B * _S, 1)
    x03 = x0.reshape(_B, 1, _N)
    x13 = x1.reshape(_B, 1, _N)
    x23 = x2.reshape(_B, 1, _N)
    idx, gidx = _bq(x03, x13, x23, c0t, c1t, c2t)
    tab = jnp.concatenate(
        [x_complete, x, jnp.zeros((_B, _N, _DF - _CIN - 3), jnp.float32)],
        axis=2).reshape(_B * _N, _DF)
    feat = _sc_gather(tab, gidx.reshape(_ROWS))
    w0p = jnp.pad(W0.T, ((0, _DF - (_CIN + 3)), (0, 0)))
    h1, st1 = _mlp1(feat, c0t, c1t, c2t, w0p, b0[None])
    h2, st2 = _mlp2(h1, st1, W1.T, b1[None], g0[None], beta0[None])
    out4, = _pool(h2, st2, g1[None], beta1[None], idx.reshape(_ROWS, 1))
    return centroids, out4.reshape(_B, _S, _D2)


# speedup vs baseline: 163.8299x; 1.2830x over previous
import functools

import numpy as np
import jax
import jax.numpy as jnp
from jax import lax
from jax.experimental import pallas as pl
from jax.experimental.pallas import tpu as pltpu
from jax.experimental.pallas import tpu_sc as plsc

_B, _N, _S, _K = 4, 8192, 1024, 32
_R2 = np.float32(0.04)
_CIN = 64
_D1, _D2 = 32, 64
_EPS = np.float32(1e-5)
_DF = 128
_ROWS = _B * _S * _K
_SBLK = 64
_RBLK = _SBLK * _K
_NBLK = (_B * _S) // _SBLK
_GCH = 128
_CCH = 1024


_NR = 8
_NC = _N // _NR


def _fps_kernel(x0_ref, x1_ref, x2_ref, c0_ref, c1_ref, c2_ref, dist_ref):
    niota = (lax.broadcasted_iota(jnp.int32, (_B, _NR, _NC), 1) * _NC
             + lax.broadcasted_iota(jnp.int32, (_B, _NR, _NC), 2))
    lane_s = lax.broadcasted_iota(jnp.int32, (_B, 1, _S), 2)
    dist_ref[...] = jnp.full((_B, _NR, _NC), 1e10, jnp.float32)

    def body(i, far):
        x0 = x0_ref[...]
        x1 = x1_ref[...]
        x2 = x2_ref[...]
        eq = niota == far
        c0 = jnp.sum(jnp.sum(jnp.where(eq, x0, 0.0), axis=2, keepdims=True),
                     axis=1, keepdims=True)
        c1 = jnp.sum(jnp.sum(jnp.where(eq, x1, 0.0), axis=2, keepdims=True),
                     axis=1, keepdims=True)
        c2 = jnp.sum(jnp.sum(jnp.where(eq, x2, 0.0), axis=2, keepdims=True),
                     axis=1, keepdims=True)
        sel = lane_s == i
        c0_ref[...] = jnp.where(sel, c0, c0_ref[...])
        c1_ref[...] = jnp.where(sel, c1, c1_ref[...])
        c2_ref[...] = jnp.where(sel, c2, c2_ref[...])
        d0 = x0 - c0
        d1 = x1 - c1
        d2 = x2 - c2
        d = (d0 * d0 + d1 * d1) + d2 * d2
        nd = jnp.minimum(dist_ref[...], d)
        dist_ref[...] = nd
        m = jnp.max(jnp.max(nd, axis=2, keepdims=True), axis=1, keepdims=True)
        cand = jnp.where(nd == m, niota, _N)
        return jnp.min(jnp.min(cand, axis=2, keepdims=True),
                       axis=1, keepdims=True)

    lax.fori_loop(0, _S, body, jnp.zeros((_B, 1, 1), jnp.int32))


def _fps(x0r, x1r, x2r):
    return pl.pallas_call(
        _fps_kernel,
        out_shape=[jax.ShapeDtypeStruct((_B, 1, _S), jnp.float32)] * 3,
        scratch_shapes=[pltpu.VMEM((_B, _NR, _NC), jnp.float32)],
    )(x0r, x1r, x2r)


def _bq_kernel(x0_ref, x1_ref, x2_ref, c0_ref, c1_ref, c2_ref,
               idx_ref, gidx_ref):
    b = pl.program_id(0)
    c0 = c0_ref[...]
    c1 = c1_ref[...]
    c2 = c2_ref[...]
    cn = (c0 * c0 + c1 * c1) + c2 * c2
    c0b = c0.astype(jnp.bfloat16).astype(jnp.float32)
    c1b = c1.astype(jnp.bfloat16).astype(jnp.float32)
    c2b = c2.astype(jnp.bfloat16).astype(jnp.float32)
    lanei = lax.broadcasted_iota(jnp.int32, (_SBLK, _CCH), 1)

    def chunk(state):
        c, rc, acc = state
        off = pl.multiple_of(c * _CCH, _CCH)
        x0 = x0_ref[0, :, pl.ds(off, _CCH)]
        x1 = x1_ref[0, :, pl.ds(off, _CCH)]
        x2 = x2_ref[0, :, pl.ds(off, _CCH)]
        xn = (x0 * x0 + x1 * x1) + x2 * x2
        x0b = x0.astype(jnp.bfloat16).astype(jnp.float32)
        x1b = x1.astype(jnp.bfloat16).astype(jnp.float32)
        x2b = x2.astype(jnp.bfloat16).astype(jnp.float32)
        dot = (c0b * x0b + c1b * x1b) + c2b * x2b
        d2 = (cn + xn) - 2.0 * dot
        pos = (d2 < _R2).astype(jnp.float32)
        shift = 1
        while shift < _CCH:
            rolled = pltpu.roll(pos, shift, 1)
            pos = pos + jnp.where(lanei >= shift, rolled, 0.0)
            shift *= 2
        pos = pos + rc
        cols = []
        for j in range(1, _K + 1):
            cols.append(jnp.sum((pos < j).astype(jnp.float32), axis=1,
                                keepdims=True))
        acc = acc + jnp.concatenate(cols, axis=1)
        return c + 1, pos[:, _CCH - 1:_CCH], acc

    def cond(state):
        c, rc, _ = state
        return jnp.logical_and(c < _N // _CCH, jnp.min(rc) < _K)

    _, _, cnt = lax.while_loop(
        cond, chunk,
        (jnp.int32(0), jnp.zeros((_SBLK, 1), jnp.float32),
         jnp.zeros((_SBLK, _K), jnp.float32)))
    iv = cnt.astype(jnp.int32)
    idxv = jnp.where(iv >= _N, -1, iv)
    idx_ref[...] = idxv[None]
    gidx_ref[...] = (jnp.maximum(idxv, 0) + b * _N)[None]


def _bq(x0, x1, x2, c0t, c1t, c2t):
    return pl.pallas_call(
        _bq_kernel,
        grid=(_B, _S // _SBLK),
        in_specs=[pl.BlockSpec((1, 1, _N), lambda b, s: (b, 0, 0))] * 3
        + [pl.BlockSpec((_SBLK, 1),
                        lambda b, s: (b * (_S // _SBLK) + s, 0))] * 3,
        out_specs=[pl.BlockSpec((1, _SBLK, _K), lambda b, s: (b, s, 0))] * 2,
        out_shape=[jax.ShapeDtypeStruct((_B, _S, _K), jnp.int32)] * 2,
        compiler_params=pltpu.CompilerParams(
            dimension_semantics=("arbitrary", "arbitrary")),
    )(x0, x1, x2, c0t, c1t, c2t)


def _sc_gather(tab, gidx):
    nw = 32
    bpw = _ROWS // nw
    mesh = plsc.VectorSubcoreMesh(core_axis_name="c", subcore_axis_name="s")

    @functools.partial(
        pl.kernel, mesh=mesh,
        out_type=jax.ShapeDtypeStruct((_ROWS, _DF), jnp.float32),
        scratch_types=[pltpu.VMEM((_GCH,), jnp.int32),
                       pltpu.VMEM((_GCH, _DF), jnp.float32),
                       pltpu.SemaphoreType.DMA],
    )
    def k(tab_hbm, idx_hbm, out_hbm, idx_v, rows_v, sem):
        wid = lax.axis_index("s") * 2 + lax.axis_index("c")
        base = wid * bpw

        @pl.loop(0, bpw // _GCH)
        def _(c):
            b0 = base + c * _GCH
            pltpu.sync_copy(idx_hbm.at[pl.ds(b0, _GCH)], idx_v)
            pltpu.async_copy(tab_hbm.at[idx_v], rows_v, sem).wait()
            pltpu.sync_copy(rows_v, out_hbm.at[pl.ds(b0, _GCH)])

    return k(tab, gidx)


def _mlp1_kernel(f_ref, c0_ref, c1_ref, c2_ref, w_ref, b_ref,
                 h_ref, st_ref, acc_ref):
    i = pl.program_id(0)
    f = f_ref[...]
    w = w_ref[...]
    h = jnp.dot(f, w, preferred_element_type=jnp.float32) + b_ref[...]
    corr = (c0_ref[...] * w[_CIN:_CIN + 1, :]
            + c1_ref[...] * w[_CIN + 1:_CIN + 2, :]
            + c2_ref[...] * w[_CIN + 2:_CIN + 3, :])
    hp = (h.reshape(_SBLK, _K, _D1) - corr[:, None, :]).reshape(_RBLK, _D1)
    h_ref[...] = hp
    s = jnp.sum(hp, axis=0, keepdims=True)
    ss = jnp.sum(hp * hp, axis=0, keepdims=True)

    @pl.when(i == 0)
    def _():
        acc_ref[...] = jnp.zeros((2, _D1), jnp.float32)

    acc_ref[...] += jnp.concatenate([s, ss], axis=0)
    st_ref[...] = acc_ref[...]


def _mlp1(feat, c0t, c1t, c2t, w0p, b0):
    return pl.pallas_call(
        _mlp1_kernel,
        grid=(_NBLK,),
        in_specs=[pl.BlockSpec((_RBLK, _DF), lambda i: (i, 0))]
        + [pl.BlockSpec((_SBLK, 1), lambda i: (i, 0))] * 3
        + [pl.BlockSpec((_DF, _D1), lambda i: (0, 0)),
           pl.BlockSpec((1, _D1), lambda i: (0, 0))],
        out_specs=[pl.BlockSpec((_RBLK, _D1), lambda i: (i, 0)),
                   pl.BlockSpec((2, _D1), lambda i: (0, 0))],
        out_shape=[jax.ShapeDtypeStruct((_ROWS, _D1), jnp.float32),
                   jax.ShapeDtypeStruct((2, _D1), jnp.float32)],
        scratch_shapes=[pltpu.VMEM((2, _D1), jnp.float32)],
        compiler_params=pltpu.CompilerParams(
            dimension_semantics=("arbitrary",)),
    )(feat, c0t, c1t, c2t, w0p, b0)


def _mlp2_kernel(h_ref, st_ref, w_ref, b_ref, g_ref, be_ref,
                 h2_ref, st2_ref, acc_ref):
    i = pl.program_id(0)
    st = st_ref[...]
    m = st[0:1, :] * (1.0 / _ROWS)
    v = st[1:2, :] * (1.0 / _ROWS) - m * m
    sc = g_ref[...] / jnp.sqrt(v + _EPS)
    sh = be_ref[...] - m * sc
    h1 = jnp.maximum(h_ref[...] * sc + sh, 0.0)
    h2 = jnp.dot(h1, w_ref[...], preferred_element_type=jnp.float32) + b_ref[...]
    h2_ref[...] = h2
    s = jnp.sum(h2, axis=0, keepdims=True)
    ss = jnp.sum(h2 * h2, axis=0, keepdims=True)

    @pl.when(i == 0)
    def _():
        acc_ref[...] = jnp.zeros((2, _D2), jnp.float32)

    acc_ref[...] += jnp.concatenate([s, ss], axis=0)
    st2_ref[...] = acc_ref[...]


def _mlp2(h1, st1, w1t, b1, g0, beta0):
    return pl.pallas_call(
        _mlp2_kernel,
        grid=(_NBLK,),
        in_specs=[pl.BlockSpec((_RBLK, _D1), lambda i: (i, 0)),
                  pl.BlockSpec((2, _D1), lambda i: (0, 0)),
                  pl.BlockSpec((_D1, _D2), lambda i: (0, 0)),
                  pl.BlockSpec((1, _D2), lambda i: (0, 0)),
                  pl.BlockSpec((1, _D1), lambda i: (0, 0)),
                  pl.BlockSpec((1, _D1), lambda i: (0, 0))],
        out_specs=[pl.BlockSpec((_RBLK, _D2), lambda i: (i, 0)),
                   pl.BlockSpec((2, _D2), lambda i: (0, 0))],
        out_shape=[jax.ShapeDtypeStruct((_ROWS, _D2), jnp.float32),
                   jax.ShapeDtypeStruct((2, _D2), jnp.float32)],
        scratch_shapes=[pltpu.VMEM((2, _D2), jnp.float32)],
        compiler_params=pltpu.CompilerParams(
            dimension_semantics=("arbitrary",)),
    )(h1, st1, w1t, b1, g0, beta0)


def _pool_kernel(h2_ref, st_ref, g_ref, be_ref, idx_ref, o_ref):
    st = st_ref[...]
    m = st[0:1, :] * (1.0 / _ROWS)
    v = st[1:2, :] * (1.0 / _ROWS) - m * m
    sc = g_ref[...] / jnp.sqrt(v + _EPS)
    sh = be_ref[...] - m * sc
    h = jnp.maximum(h2_ref[...] * sc + sh, 0.0)
    msk = idx_ref[...] == -1
    hm = jnp.where(msk, -jnp.inf, h)
    o_ref[...] = jnp.max(hm.reshape(_SBLK, _K, _D2), axis=1)


def _pool(h2, st2, g1, beta1, idx2):
    return pl.pallas_call(
        _pool_kernel,
        grid=(_NBLK,),
        in_specs=[pl.BlockSpec((_RBLK, _D2), lambda i: (i, 0)),
                  pl.BlockSpec((2, _D2), lambda i: (0, 0)),
                  pl.BlockSpec((1, _D2), lambda i: (0, 0)),
                  pl.BlockSpec((1, _D2), lambda i: (0, 0)),
                  pl.BlockSpec((_RBLK, 1), lambda i: (i, 0))],
        out_specs=[pl.BlockSpec((_SBLK, _D2), lambda i: (i, 0))],
        out_shape=[jax.ShapeDtypeStruct((_B * _S, _D2), jnp.float32)],
        compiler_params=pltpu.CompilerParams(
            dimension_semantics=("arbitrary",)),
    )(h2, st2, g1, beta1, idx2)


def kernel(x, x_complete, W0, b0, g0, beta0, W1, b1, g1, beta1):
    x0 = x[:, :, 0]
    x1 = x[:, :, 1]
    x2 = x[:, :, 2]
    c03, c13, c23 = _fps(x0.reshape(_B, _NR, _NC), x1.reshape(_B, _NR, _NC),
                         x2.reshape(_B, _NR, _NC))
    c0 = c03.reshape(_B, _S)
    c1 = c13.reshape(_B, _S)
    c2 = c23.reshape(_B, _S)
    centroids = jnp.stack([c0, c1, c2], axis=-1)
    c0t = c0.reshape(_B * _S, 1)
    c1t = c1.reshape(_B * _S, 1)
    c2t = c2.reshape(_B * _S, 1)
    x03 = x0.reshape(_B, 1, _N)
    x13 = x1.reshape(_B, 1, _N)
    x23 = x2.reshape(_B, 1, _N)
    idx, gidx = _bq(x03, x13, x23, c0t, c1t, c2t)
    tab = jnp.concatenate(
        [x_complete, x, jnp.zeros((_B, _N, _DF - _CIN - 3), jnp.float32)],
        axis=2).reshape(_B * _N, _DF)
    feat = _sc_gather(tab, gidx.reshape(_ROWS))
    w0p = jnp.pad(W0.T, ((0, _DF - (_CIN + 3)), (0, 0)))
    h1, st1 = _mlp1(feat, c0t, c1t, c2t, w0p, b0[None])
    h2, st2 = _mlp2(h1, st1, W1.T, b1[None], g0[None], beta0[None])
    out4, = _pool(h2, st2, g1[None], beta1[None], idx.reshape(_ROWS, 1))
    return centroids, out4.reshape(_B, _S, _D2)


# speedup vs baseline: 173.4008x; 1.0584x over previous
import functools

import numpy as np
import jax
import jax.numpy as jnp
from jax import lax
from jax.experimental import pallas as pl
from jax.experimental.pallas import tpu as pltpu
from jax.experimental.pallas import tpu_sc as plsc

_B, _N, _S, _K = 4, 8192, 1024, 32
_R2 = np.float32(0.04)
_CIN = 64
_D1, _D2 = 32, 64
_EPS = np.float32(1e-5)
_DF = 128
_ROWS = _B * _S * _K
_SBLK = 64
_RBLK = _SBLK * _K
_NBLK = (_B * _S) // _SBLK
_GCH = 128
_CCH = 1024


_NR = 8
_NC = _N // _NR


def _fps_kernel(x0_ref, x1_ref, x2_ref, c0_ref, c1_ref, c2_ref, dist_ref):
    niota = (lax.broadcasted_iota(jnp.int32, (_B, _NR, _NC), 1) * _NC
             + lax.broadcasted_iota(jnp.int32, (_B, _NR, _NC), 2))
    lane_s = lax.broadcasted_iota(jnp.int32, (_B, 1, _S), 2)
    dist_ref[...] = jnp.full((_B, _NR, _NC), 1e10, jnp.float32)

    def body(i, far):
        x0 = x0_ref[...]
        x1 = x1_ref[...]
        x2 = x2_ref[...]
        eq = niota == far
        c0 = jnp.sum(jnp.sum(jnp.where(eq, x0, 0.0), axis=2, keepdims=True),
                     axis=1, keepdims=True)
        c1 = jnp.sum(jnp.sum(jnp.where(eq, x1, 0.0), axis=2, keepdims=True),
                     axis=1, keepdims=True)
        c2 = jnp.sum(jnp.sum(jnp.where(eq, x2, 0.0), axis=2, keepdims=True),
                     axis=1, keepdims=True)
        sel = lane_s == i
        c0_ref[...] = jnp.where(sel, c0, c0_ref[...])
        c1_ref[...] = jnp.where(sel, c1, c1_ref[...])
        c2_ref[...] = jnp.where(sel, c2, c2_ref[...])
        d0 = x0 - c0
        d1 = x1 - c1
        d2 = x2 - c2
        d = (d0 * d0 + d1 * d1) + d2 * d2
        nd = jnp.minimum(dist_ref[...], d)
        dist_ref[...] = nd
        m = jnp.max(jnp.max(nd, axis=2, keepdims=True), axis=1, keepdims=True)
        cand = jnp.where(nd == m, niota, _N)
        return jnp.min(jnp.min(cand, axis=2, keepdims=True),
                       axis=1, keepdims=True)

    lax.fori_loop(0, _S, body, jnp.zeros((_B, 1, 1), jnp.int32))


def _fps(x0r, x1r, x2r):
    return pl.pallas_call(
        _fps_kernel,
        out_shape=[jax.ShapeDtypeStruct((_B, 1, _S), jnp.float32)] * 3,
        scratch_shapes=[pltpu.VMEM((_B, _NR, _NC), jnp.float32)],
    )(x0r, x1r, x2r)


def _bq_kernel(x0_ref, x1_ref, x2_ref, c0_ref, c1_ref, c2_ref,
               idx_ref, gidx_ref):
    b = pl.program_id(0)
    c0 = c0_ref[...]
    c1 = c1_ref[...]
    c2 = c2_ref[...]
    cn = (c0 * c0 + c1 * c1) + c2 * c2
    c0b = c0.astype(jnp.bfloat16).astype(jnp.float32)
    c1b = c1.astype(jnp.bfloat16).astype(jnp.float32)
    c2b = c2.astype(jnp.bfloat16).astype(jnp.float32)
    lanei = lax.broadcasted_iota(jnp.int32, (_SBLK, _CCH), 1)

    def chunk(state):
        c, rc, acc = state
        off = pl.multiple_of(c * _CCH, _CCH)
        x0 = x0_ref[0, :, pl.ds(off, _CCH)]
        x1 = x1_ref[0, :, pl.ds(off, _CCH)]
        x2 = x2_ref[0, :, pl.ds(off, _CCH)]
        xn = (x0 * x0 + x1 * x1) + x2 * x2
        x0b = x0.astype(jnp.bfloat16).astype(jnp.float32)
        x1b = x1.astype(jnp.bfloat16).astype(jnp.float32)
        x2b = x2.astype(jnp.bfloat16).astype(jnp.float32)
        dot = (c0b * x0b + c1b * x1b) + c2b * x2b
        d2 = (cn + xn) - 2.0 * dot
        pos = (d2 < _R2).astype(jnp.float32)
        shift = 1
        while shift < _CCH:
            rolled = pltpu.roll(pos, shift, 1)
            pos = pos + jnp.where(lanei >= shift, rolled, 0.0)
            shift *= 2
        pos = pos + rc
        cols = []
        for j in range(1, _K + 1):
            cols.append(jnp.sum((pos < j).astype(jnp.float32), axis=1,
                                keepdims=True))
        acc = acc + jnp.concatenate(cols, axis=1)
        return c + 1, pos[:, _CCH - 1:_CCH], acc

    def cond(state):
        c, rc, _ = state
        return jnp.logical_and(c < _N // _CCH, jnp.min(rc) < _K)

    _, _, cnt = lax.while_loop(
        cond, chunk,
        (jnp.int32(0), jnp.zeros((_SBLK, 1), jnp.float32),
         jnp.zeros((_SBLK, _K), jnp.float32)))
    iv = cnt.astype(jnp.int32)
    idxv = jnp.where(iv >= _N, -1, iv)
    idx_ref[...] = idxv[None]
    gidx_ref[...] = (jnp.maximum(idxv, 0) + b * _N)[None]


def _bq(x0, x1, x2, c0t, c1t, c2t):
    return pl.pallas_call(
        _bq_kernel,
        grid=(_B, _S // _SBLK),
        in_specs=[pl.BlockSpec((1, 1, _N), lambda b, s: (b, 0, 0))] * 3
        + [pl.BlockSpec((_SBLK, 1),
                        lambda b, s: (b * (_S // _SBLK) + s, 0))] * 3,
        out_specs=[pl.BlockSpec((1, _SBLK, _K), lambda b, s: (b, s, 0))] * 2,
        out_shape=[jax.ShapeDtypeStruct((_B, _S, _K), jnp.int32)] * 2,
        compiler_params=pltpu.CompilerParams(
            dimension_semantics=("arbitrary", "arbitrary")),
    )(x0, x1, x2, c0t, c1t, c2t)


def _sc_gather(tab, gidx):
    nw = 32
    bpw = _ROWS // nw
    mesh = plsc.VectorSubcoreMesh(core_axis_name="c", subcore_axis_name="s")

    @functools.partial(
        pl.kernel, mesh=mesh,
        out_type=jax.ShapeDtypeStruct((_ROWS, _DF), jnp.float32),
        scratch_types=[pltpu.VMEM((bpw,), jnp.int32),
                       pltpu.VMEM((_GCH, _DF), jnp.float32),
                       pltpu.VMEM((_GCH, _DF), jnp.float32),
                       pltpu.SemaphoreType.DMA,
                       pltpu.SemaphoreType.DMA],
    )
    def k(tab_hbm, idx_hbm, out_hbm, idx_v, rows0, rows1, sem0, sem1):
        wid = lax.axis_index("s") * 2 + lax.axis_index("c")
        base = wid * bpw
        pltpu.sync_copy(idx_hbm.at[pl.ds(base, bpw)], idx_v)

        @pl.loop(0, bpw // _GCH, step=2)
        def _(cc):
            c = cc * _GCH
            g0 = pltpu.async_copy(
                tab_hbm.at[idx_v.at[pl.ds(c, _GCH)]], rows0, sem0)
            g1 = pltpu.async_copy(
                tab_hbm.at[idx_v.at[pl.ds(c + _GCH, _GCH)]], rows1, sem1)
            g0.wait()
            pltpu.sync_copy(rows0, out_hbm.at[pl.ds(base + c, _GCH)])
            g1.wait()
            pltpu.sync_copy(rows1, out_hbm.at[pl.ds(base + c + _GCH, _GCH)])

    return k(tab, gidx)


def _mlp1_kernel(f_ref, c0_ref, c1_ref, c2_ref, w_ref, b_ref,
                 h_ref, st_ref, acc_ref):
    i = pl.program_id(0)
    f = f_ref[...]
    w = w_ref[...]
    h = jnp.dot(f.astype(jnp.bfloat16), w.astype(jnp.bfloat16),
                preferred_element_type=jnp.float32) + b_ref[...]
    corr = (c0_ref[...] * w[_CIN:_CIN + 1, :]
            + c1_ref[...] * w[_CIN + 1:_CIN + 2, :]
            + c2_ref[...] * w[_CIN + 2:_CIN + 3, :])
    hp = (h.reshape(_SBLK, _K, _D1) - corr[:, None, :]).reshape(_RBLK, _D1)
    h_ref[...] = hp
    s = jnp.sum(hp, axis=0, keepdims=True)
    ss = jnp.sum(hp * hp, axis=0, keepdims=True)

    @pl.when(i == 0)
    def _():
        acc_ref[...] = jnp.zeros((2, _D1), jnp.float32)

    acc_ref[...] += jnp.concatenate([s, ss], axis=0)
    st_ref[...] = acc_ref[...]


def _mlp1(feat, c0t, c1t, c2t, w0p, b0):
    return pl.pallas_call(
        _mlp1_kernel,
        grid=(_NBLK,),
        in_specs=[pl.BlockSpec((_RBLK, _DF), lambda i: (i, 0))]
        + [pl.BlockSpec((_SBLK, 1), lambda i: (i, 0))] * 3
        + [pl.BlockSpec((_DF, _D1), lambda i: (0, 0)),
           pl.BlockSpec((1, _D1), lambda i: (0, 0))],
        out_specs=[pl.BlockSpec((_RBLK, _D1), lambda i: (i, 0)),
                   pl.BlockSpec((2, _D1), lambda i: (0, 0))],
        out_shape=[jax.ShapeDtypeStruct((_ROWS, _D1), jnp.float32),
                   jax.ShapeDtypeStruct((2, _D1), jnp.float32)],
        scratch_shapes=[pltpu.VMEM((2, _D1), jnp.float32)],
        compiler_params=pltpu.CompilerParams(
            dimension_semantics=("arbitrary",)),
    )(feat, c0t, c1t, c2t, w0p, b0)


def _mlp2_kernel(h_ref, st_ref, w_ref, b_ref, g_ref, be_ref, idx_ref,
                 hm_ref, st2_ref, acc_ref):
    i = pl.program_id(0)
    st = st_ref[...]
    m = st[0:1, :] * (1.0 / _ROWS)
    v = st[1:2, :] * (1.0 / _ROWS) - m * m
    sc = g_ref[...] / jnp.sqrt(v + _EPS)
    sh = be_ref[...] - m * sc
    h1 = jnp.maximum(h_ref[...] * sc + sh, 0.0)
    h2 = jnp.dot(h1.astype(jnp.bfloat16), w_ref[...].astype(jnp.bfloat16),
                 preferred_element_type=jnp.float32) + b_ref[...]
    s = jnp.sum(h2, axis=0, keepdims=True)
    ss = jnp.sum(h2 * h2, axis=0, keepdims=True)

    @pl.when(i == 0)
    def _():
        acc_ref[...] = jnp.zeros((2, _D2), jnp.float32)

    acc_ref[...] += jnp.concatenate([s, ss], axis=0)
    st2_ref[...] = acc_ref[...]
    msk = idx_ref[...] == -1
    hm = jnp.where(msk, -jnp.inf, h2)
    hm_ref[...] = jnp.max(hm.reshape(_SBLK, _K, _D2), axis=1)


def _mlp2(h1, st1, w1t, b1, g0, beta0, idxr):
    return pl.pallas_call(
        _mlp2_kernel,
        grid=(_NBLK,),
        in_specs=[pl.BlockSpec((_RBLK, _D1), lambda i: (i, 0)),
                  pl.BlockSpec((2, _D1), lambda i: (0, 0)),
                  pl.BlockSpec((_D1, _D2), lambda i: (0, 0)),
                  pl.BlockSpec((1, _D2), lambda i: (0, 0)),
                  pl.BlockSpec((1, _D1), lambda i: (0, 0)),
                  pl.BlockSpec((1, _D1), lambda i: (0, 0)),
                  pl.BlockSpec((_RBLK, 1), lambda i: (i, 0))],
        out_specs=[pl.BlockSpec((_SBLK, _D2), lambda i: (i, 0)),
                   pl.BlockSpec((2, _D2), lambda i: (0, 0))],
        out_shape=[jax.ShapeDtypeStruct((_B * _S, _D2), jnp.float32),
                   jax.ShapeDtypeStruct((2, _D2), jnp.float32)],
        scratch_shapes=[pltpu.VMEM((2, _D2), jnp.float32)],
        compiler_params=pltpu.CompilerParams(
            dimension_semantics=("arbitrary",)),
    )(h1, st1, w1t, b1, g0, beta0, idxr)


def _pool_kernel(hm_ref, st_ref, g_ref, be_ref, o_ref):
    st = st_ref[...]
    m = st[0:1, :] * (1.0 / _ROWS)
    v = st[1:2, :] * (1.0 / _ROWS) - m * m
    sc = g_ref[...] / jnp.sqrt(v + _EPS)
    sh = be_ref[...] - m * sc
    o_ref[...] = jnp.maximum(hm_ref[...] * sc + sh, 0.0)


def _pool(hmax, st2, g1, beta1):
    return pl.pallas_call(
        _pool_kernel,
        out_shape=[jax.ShapeDtypeStruct((_B * _S, _D2), jnp.float32)],
    )(hmax, st2, g1, beta1)


def kernel(x, x_complete, W0, b0, g0, beta0, W1, b1, g1, beta1):
    x0 = x[:, :, 0]
    x1 = x[:, :, 1]
    x2 = x[:, :, 2]
    c03, c13, c23 = _fps(x0.reshape(_B, _NR, _NC), x1.reshape(_B, _NR, _NC),
                         x2.reshape(_B, _NR, _NC))
    c0 = c03.reshape(_B, _S)
    c1 = c13.reshape(_B, _S)
    c2 = c23.reshape(_B, _S)
    centroids = jnp.stack([c0, c1, c2], axis=-1)
    c0t = c0.reshape(_B * _S, 1)
    c1t = c1.reshape(_B * _S, 1)
    c2t = c2.reshape(_B * _S, 1)
    x03 = x0.reshape(_B, 1, _N)
    x13 = x1.reshape(_B, 1, _N)
    x23 = x2.reshape(_B, 1, _N)
    idx, gidx = _bq(x03, x13, x23, c0t, c1t, c2t)
    tab = jnp.concatenate(
        [x_complete, x, jnp.zeros((_B, _N, _DF - _CIN - 3), jnp.float32)],
        axis=2).reshape(_B * _N, _DF)
    feat = _sc_gather(tab, gidx.reshape(_ROWS))
    w0p = jnp.pad(W0.T, ((0, _DF - (_CIN + 3)), (0, 0)))
    h1, st1 = _mlp1(feat, c0t, c1t, c2t, w0p, b0[None])
    hmax, st2 = _mlp2(h1, st1, W1.T, b1[None], g0[None], beta0[None],
                      idx.reshape(_ROWS, 1))
    out4, = _pool(hmax, st2, g1[None], beta1[None])
    return centroids, out4.reshape(_B, _S, _D2)
